# Initial kernel scaffold; baseline (speedup 1.0000x reference)
import jax
import jax.numpy as jnp
from jax.experimental import pallas as pl


def kernel(union_indices, o_positions, d_positions, position_encoding, iso_o, iso_d, grid_features, W_f, b_f, W_p, b_p):
    raise NotImplementedError("write your pallas kernel here")


# speedup vs baseline: 6.0104x; 6.0104x over previous
import functools

import jax
import jax.numpy as jnp
from jax import lax
from jax.experimental import pallas as pl
from jax.experimental.pallas import tpu as pltpu
from jax.experimental.pallas import tpu_sc as plsc

B = 1024
N_UNION = 200
N_OD = 100
FEAT = 32
POS = 8
EMB = 64

NC = 2
NS = 16
NW = NC * NS
BPW = B // NW
LANES = 16
NPAD = 112
NVJ = NPAD // LANES


def _sc_gather_build():
    mesh = plsc.VectorSubcoreMesh(core_axis_name="c", subcore_axis_name="s")

    @functools.partial(
        pl.kernel,
        out_type=(
            jax.ShapeDtypeStruct((B, N_OD, FEAT), jnp.float32),
            jax.ShapeDtypeStruct((B, N_OD, FEAT), jnp.float32),
            jax.ShapeDtypeStruct((B, N_OD, POS), jnp.float32),
            jax.ShapeDtypeStruct((B, N_OD, POS), jnp.float32),
        ),
        mesh=mesh,
        compiler_params=pltpu.CompilerParams(
            needs_layout_passes=False, use_tc_tiling_on_sc=False),
        scratch_types=[
            pltpu.VMEM((BPW * N_UNION,), jnp.int32),
            pltpu.VMEM((BPW, 2, NPAD), jnp.int32),
            pltpu.VMEM((BPW, 2, NPAD), jnp.int32),
            pltpu.VMEM((BPW, 2, NPAD), jnp.int32),
            pltpu.VMEM((2, NPAD, FEAT), jnp.float32),
            pltpu.VMEM((2, NPAD, POS), jnp.float32),
            pltpu.SemaphoreType.DMA,
        ],
    )
    def sc_gather(union_hbm, pos_hbm, peflat_hbm, grid_hbm,
                  rows_o_hbm, rows_d_hbm, pe_o_hbm, pe_d_hbm,
                  union_v, pos_v, gidx_v, peidx_v, rows_v, pev, sem):
        wid = lax.axis_index("s") * NC + lax.axis_index("c")
        b0 = wid * BPW

        pltpu.sync_copy(union_hbm.at[pl.ds(b0 * N_UNION, BPW * N_UNION)], union_v)
        pltpu.sync_copy(pos_hbm.at[pl.ds(b0, BPW)], pos_v)

        def compose(i, _):
            for c in range(2):
                for j in range(NVJ):
                    pv = pos_v[i, c, pl.ds(j * LANES, LANES)]
                    u = plsc.load_gather(union_v, [pv + i * N_UNION])
                    gidx_v[i, c, pl.ds(j * LANES, LANES)] = u
                    peidx_v[i, c, pl.ds(j * LANES, LANES)] = (
                        pv + (b0 + i) * N_UNION)
            return 0

        lax.fori_loop(0, BPW, compose, 0)

        def gather_one(i, _):
            b = b0 + i
            cps = [
                pltpu.async_copy(grid_hbm.at[gidx_v.at[i, 0]], rows_v.at[0], sem),
                pltpu.async_copy(grid_hbm.at[gidx_v.at[i, 1]], rows_v.at[1], sem),
                pltpu.async_copy(peflat_hbm.at[peidx_v.at[i, 0]], pev.at[0], sem),
                pltpu.async_copy(peflat_hbm.at[peidx_v.at[i, 1]], pev.at[1], sem),
            ]
            for cp in cps:
                cp.wait()
            pltpu.sync_copy(rows_v.at[0, pl.ds(0, N_OD)], rows_o_hbm.at[b])
            pltpu.sync_copy(rows_v.at[1, pl.ds(0, N_OD)], rows_d_hbm.at[b])
            pltpu.sync_copy(pev.at[0, pl.ds(0, N_OD)], pe_o_hbm.at[b])
            pltpu.sync_copy(pev.at[1, pl.ds(0, N_OD)], pe_d_hbm.at[b])
            return 0

        lax.fori_loop(0, BPW, gather_one, 0)

    return sc_gather


@functools.lru_cache(maxsize=1)
def _sc_gather():
    return _sc_gather_build()


BB = 32


def _tc_body(rows_o_ref, rows_d_ref, pe_o_ref, pe_d_ref, iso_o_ref, iso_d_ref,
             wf_ref, bf_ref, wp_ref, bp_ref, out_o_ref, out_d_ref):
    wf = wf_ref[...]
    bf = bf_ref[...]
    wp = wp_ref[...]
    bp = bp_ref[...]

    def emb(rows_ref, pe_ref, iso_ref, out_ref):
        r = rows_ref[...].reshape(BB * N_OD, FEAT)
        p = pe_ref[...].reshape(BB * N_OD, POS)
        f = jnp.maximum(jnp.dot(r, wf, preferred_element_type=jnp.float32) + bf, 0.0)
        q = jnp.maximum(jnp.dot(p, wp, preferred_element_type=jnp.float32) + bp, 0.0)
        out_ref[...] = (f + q).reshape(BB, N_OD, EMB) + iso_ref[...]

    emb(rows_o_ref, pe_o_ref, iso_o_ref, out_o_ref)
    emb(rows_d_ref, pe_d_ref, iso_d_ref, out_d_ref)


@functools.lru_cache(maxsize=1)
def _tc_mlp():
    bspec3 = lambda n, k: pl.BlockSpec((BB, n, k), lambda i: (i, 0, 0))
    wspec = lambda a, b: pl.BlockSpec((a, b), lambda i: (0, 0))
    return pl.pallas_call(
        _tc_body,
        grid=(B // BB,),
        in_specs=[
            bspec3(N_OD, FEAT), bspec3(N_OD, FEAT),
            bspec3(N_OD, POS), bspec3(N_OD, POS),
            bspec3(N_OD, EMB), bspec3(N_OD, EMB),
            wspec(FEAT, EMB), wspec(1, EMB), wspec(POS, EMB), wspec(1, EMB),
        ],
        out_specs=[bspec3(N_OD, EMB), bspec3(N_OD, EMB)],
        out_shape=[
            jax.ShapeDtypeStruct((B, N_OD, EMB), jnp.float32),
            jax.ShapeDtypeStruct((B, N_OD, EMB), jnp.float32),
        ],
        compiler_params=pltpu.CompilerParams(
            dimension_semantics=("parallel",)),
    )


def kernel(union_indices, o_positions, d_positions, position_encoding,
           iso_o, iso_d, grid_features, W_f, b_f, W_p, b_p):
    union_flat = union_indices.reshape(B * N_UNION)
    pos = jnp.stack([o_positions, d_positions], axis=1)
    pos_pad = jnp.pad(pos, ((0, 0), (0, 0), (0, NPAD - N_OD)))
    peflat = position_encoding.reshape(B * N_UNION, POS)

    rows_o, rows_d, pe_o, pe_d = _sc_gather()(
        union_flat, pos_pad, peflat, grid_features)

    out_o, out_d = _tc_mlp()(
        rows_o, rows_d, pe_o, pe_d, iso_o, iso_d,
        W_f, b_f.reshape(1, EMB), W_p, b_p.reshape(1, EMB))
    return (out_o, out_d)


# speedup vs baseline: 7.1256x; 1.1855x over previous
import functools

import jax
import jax.numpy as jnp
from jax import lax
from jax.experimental import pallas as pl
from jax.experimental.pallas import tpu as pltpu
from jax.experimental.pallas import tpu_sc as plsc

B = 1024
N_UNION = 200
N_OD = 100
FEAT = 32
POS = 8
EMB = 64

NC = 2
NS = 16
NW = NC * NS
BPW = B // NW
LANES = 16
CHUNK = 128
SLOTS = 4
NCHUNK = N_OD // SLOTS


def _sc_gather_build():
    mesh = plsc.VectorSubcoreMesh(core_axis_name="c", subcore_axis_name="s")

    @functools.partial(
        pl.kernel,
        out_type=(
            jax.ShapeDtypeStruct((NW, NCHUNK, CHUNK, FEAT), jnp.float32),
            jax.ShapeDtypeStruct((NW, NCHUNK, CHUNK, FEAT), jnp.float32),
            jax.ShapeDtypeStruct((NW, NCHUNK, CHUNK, POS), jnp.float32),
            jax.ShapeDtypeStruct((NW, NCHUNK, CHUNK, POS), jnp.float32),
        ),
        mesh=mesh,
        compiler_params=pltpu.CompilerParams(
            needs_layout_passes=False, use_tc_tiling_on_sc=False),
        scratch_types=[
            pltpu.VMEM((N_UNION, BPW), jnp.int32),
            pltpu.VMEM((2, N_OD, BPW), jnp.int32),
            pltpu.VMEM((2, N_OD * BPW), jnp.int32),
            pltpu.VMEM((2, N_OD * BPW), jnp.int32),
            pltpu.VMEM((2, CHUNK, FEAT), jnp.float32),
            pltpu.VMEM((2, CHUNK, POS), jnp.float32),
            pltpu.SemaphoreType.DMA,
        ],
    )
    def sc_gather(union_hbm, opos_hbm, dpos_hbm, peflat_hbm, grid_hbm,
                  rows_o_hbm, rows_d_hbm, pe_o_hbm, pe_d_hbm,
                  union_v, pos_v, gidx_v, peidx_v, rows_v, pev, sem):
        wid = lax.axis_index("s") * NC + lax.axis_index("c")
        b0 = wid * BPW

        pltpu.sync_copy(union_hbm.at[:, pl.ds(b0, BPW)], union_v)
        pltpu.sync_copy(opos_hbm.at[:, pl.ds(b0, BPW)], pos_v.at[0])
        pltpu.sync_copy(dpos_hbm.at[:, pl.ds(b0, BPW)], pos_v.at[1])

        def compose(i, _):
            for e in range(2):
                for h in range(BPW // LANES):
                    db = h * LANES + lax.broadcasted_iota(jnp.int32, (LANES,), 0)
                    pv = pos_v[e, i, pl.ds(h * LANES, LANES)]
                    u = plsc.load_gather(union_v, [pv, db])
                    fl = i * BPW + h * LANES
                    gidx_v[e, pl.ds(fl, LANES)] = u
                    peidx_v[e, pl.ds(fl, LANES)] = (b0 + db) * N_UNION + pv
            return 0

        lax.fori_loop(0, N_OD, compose, 0)

        def gather_one(c, _):
            cps = [
                pltpu.async_copy(
                    grid_hbm.at[gidx_v.at[0, pl.ds(c * CHUNK, CHUNK)]],
                    rows_v.at[0], sem),
                pltpu.async_copy(
                    grid_hbm.at[gidx_v.at[1, pl.ds(c * CHUNK, CHUNK)]],
                    rows_v.at[1], sem),
                pltpu.async_copy(
                    peflat_hbm.at[peidx_v.at[0, pl.ds(c * CHUNK, CHUNK)]],
                    pev.at[0], sem),
                pltpu.async_copy(
                    peflat_hbm.at[peidx_v.at[1, pl.ds(c * CHUNK, CHUNK)]],
                    pev.at[1], sem),
            ]
            for cp in cps:
                cp.wait()
            pltpu.sync_copy(rows_v.at[0], rows_o_hbm.at[wid, c])
            pltpu.sync_copy(rows_v.at[1], rows_d_hbm.at[wid, c])
            pltpu.sync_copy(pev.at[0], pe_o_hbm.at[wid, c])
            pltpu.sync_copy(pev.at[1], pe_d_hbm.at[wid, c])
            return 0

        lax.fori_loop(0, NCHUNK, gather_one, 0)

    return sc_gather


@functools.lru_cache(maxsize=1)
def _sc_gather():
    return _sc_gather_build()


def _tc_body(ro_ref, rd_ref, po_ref, pd_ref, io_ref, id_ref,
             wf_ref, bf_ref, wp_ref, bp_ref, oo_ref, od_ref):
    wf = wf_ref[...]
    bf = bf_ref[...]
    wp = wp_ref[...]
    bp = bp_ref[...]
    dn = (((0,), (1,)), ((), ()))

    for slot in range(SLOTS):
        for r_ref, p_ref, i_ref, out_ref in (
                (ro_ref, po_ref, io_ref, oo_ref),
                (rd_ref, pd_ref, id_ref, od_ref)):
            x = r_ref[:, 0, pl.ds(slot * BPW, BPW), :].reshape(B, FEAT)
            pe = p_ref[:, 0, pl.ds(slot * BPW, BPW), :].reshape(B, POS)
            f = lax.dot_general(wf, x, dn, preferred_element_type=jnp.float32)
            q = lax.dot_general(wp, pe, dn, preferred_element_type=jnp.float32)
            out_ref[slot] = (jnp.maximum(f + bf, 0.0)
                             + jnp.maximum(q + bp, 0.0) + i_ref[slot])


@functools.lru_cache(maxsize=1)
def _tc_mlp():
    rspec = lambda k: pl.BlockSpec((NW, 1, CHUNK, k), lambda j: (0, j, 0, 0))
    ispec = pl.BlockSpec((SLOTS, EMB, B), lambda j: (j, 0, 0))
    wspec = lambda a, b: pl.BlockSpec((a, b), lambda j: (0, 0))
    return pl.pallas_call(
        _tc_body,
        grid=(NCHUNK,),
        in_specs=[
            rspec(FEAT), rspec(FEAT), rspec(POS), rspec(POS),
            ispec, ispec,
            wspec(FEAT, EMB), wspec(EMB, 1), wspec(POS, EMB), wspec(EMB, 1),
        ],
        out_specs=[ispec, ispec],
        out_shape=[
            jax.ShapeDtypeStruct((N_OD, EMB, B), jnp.float32),
            jax.ShapeDtypeStruct((N_OD, EMB, B), jnp.float32),
        ],
        compiler_params=pltpu.CompilerParams(
            dimension_semantics=("parallel",)),
    )


def kernel(union_indices, o_positions, d_positions, position_encoding,
           iso_o, iso_d, grid_features, W_f, b_f, W_p, b_p):
    union_t = jnp.transpose(union_indices)
    opos_t = jnp.transpose(o_positions)
    dpos_t = jnp.transpose(d_positions)
    peflat = position_encoding.reshape(B * N_UNION, POS)

    rows_o, rows_d, pe_o, pe_d = _sc_gather()(
        union_t, opos_t, dpos_t, peflat, grid_features)

    iso_ot = jnp.transpose(iso_o, (1, 2, 0))
    iso_dt = jnp.transpose(iso_d, (1, 2, 0))
    oo, od = _tc_mlp()(
        rows_o, rows_d, pe_o, pe_d, iso_ot, iso_dt,
        W_f, b_f.reshape(EMB, 1), W_p, b_p.reshape(EMB, 1))
    return (jnp.transpose(oo, (2, 0, 1)), jnp.transpose(od, (2, 0, 1)))


# speedup vs baseline: 9.4059x; 1.3200x over previous
import functools

import jax
import jax.numpy as jnp
from jax import lax
from jax.experimental import pallas as pl
from jax.experimental.pallas import tpu as pltpu
from jax.experimental.pallas import tpu_sc as plsc

B = 1024
N_UNION = 200
N_OD = 100
FEAT = 32
POS = 8
EMB = 64

NC = 2
NS = 16
NW = NC * NS
BPW = B // NW
LANES = 16
CHUNK = 128
SLOTS = 4
NCHUNK = N_OD // SLOTS


def _sc_gather_build():
    mesh = plsc.VectorSubcoreMesh(core_axis_name="c", subcore_axis_name="s")

    @functools.partial(
        pl.kernel,
        out_type=(
            jax.ShapeDtypeStruct((NW, NCHUNK, CHUNK, FEAT), jnp.float32),
            jax.ShapeDtypeStruct((NW, NCHUNK, CHUNK, FEAT), jnp.float32),
            jax.ShapeDtypeStruct((NW, NCHUNK, CHUNK, POS), jnp.float32),
            jax.ShapeDtypeStruct((NW, NCHUNK, CHUNK, POS), jnp.float32),
        ),
        mesh=mesh,
        compiler_params=pltpu.CompilerParams(
            needs_layout_passes=False, use_tc_tiling_on_sc=False),
        scratch_types=[
            pltpu.VMEM((N_UNION, BPW), jnp.int32),
            pltpu.VMEM((2, N_OD, BPW), jnp.int32),
            pltpu.VMEM((2, N_OD * BPW), jnp.int32),
            pltpu.VMEM((2, N_OD * BPW), jnp.int32),
            pltpu.VMEM((2, CHUNK, FEAT), jnp.float32),
            pltpu.VMEM((2, CHUNK, POS), jnp.float32),
            pltpu.SemaphoreType.DMA,
        ],
    )
    def sc_gather(union_hbm, opos_hbm, dpos_hbm, peflat_hbm, grid_hbm,
                  rows_o_hbm, rows_d_hbm, pe_o_hbm, pe_d_hbm,
                  union_v, pos_v, gidx_v, peidx_v, rows_v, pev, sem):
        wid = lax.axis_index("s") * NC + lax.axis_index("c")
        b0 = wid * BPW

        pltpu.sync_copy(union_hbm.at[:, pl.ds(b0, BPW)], union_v)
        pltpu.sync_copy(opos_hbm.at[:, pl.ds(b0, BPW)], pos_v.at[0])
        pltpu.sync_copy(dpos_hbm.at[:, pl.ds(b0, BPW)], pos_v.at[1])

        def compose(i, _):
            for e in range(2):
                for h in range(BPW // LANES):
                    db = h * LANES + lax.broadcasted_iota(jnp.int32, (LANES,), 0)
                    pv = pos_v[e, i, pl.ds(h * LANES, LANES)]
                    u = plsc.load_gather(union_v, [pv, db])
                    u = ((u & ~(TR_NB - 1)) | ((u & (TR_NB // 4 - 1)) << 2)
                         | ((u >> 11) & 3))
                    fl = i * BPW + h * LANES
                    gidx_v[e, pl.ds(fl, LANES)] = u
                    peidx_v[e, pl.ds(fl, LANES)] = (b0 + db) * N_UNION + pv
            return 0

        lax.fori_loop(0, N_OD, compose, 0)

        def gather_one(c, _):
            cps = [
                pltpu.async_copy(
                    grid_hbm.at[gidx_v.at[0, pl.ds(c * CHUNK, CHUNK)]],
                    rows_v.at[0], sem),
                pltpu.async_copy(
                    grid_hbm.at[gidx_v.at[1, pl.ds(c * CHUNK, CHUNK)]],
                    rows_v.at[1], sem),
                pltpu.async_copy(
                    peflat_hbm.at[peidx_v.at[0, pl.ds(c * CHUNK, CHUNK)]],
                    pev.at[0], sem),
                pltpu.async_copy(
                    peflat_hbm.at[peidx_v.at[1, pl.ds(c * CHUNK, CHUNK)]],
                    pev.at[1], sem),
            ]
            for cp in cps:
                cp.wait()
            pltpu.sync_copy(rows_v.at[0], rows_o_hbm.at[wid, c])
            pltpu.sync_copy(rows_v.at[1], rows_d_hbm.at[wid, c])
            pltpu.sync_copy(pev.at[0], pe_o_hbm.at[wid, c])
            pltpu.sync_copy(pev.at[1], pe_d_hbm.at[wid, c])
            return 0

        lax.fori_loop(0, NCHUNK, gather_one, 0)

    return sc_gather


@functools.lru_cache(maxsize=1)
def _sc_gather():
    return _sc_gather_build()


GRID_N = 1000000
TR_NB = 8192
TR_STEPS = -(-GRID_N // TR_NB)


def _tr_body(gt_ref, out_ref):
    x = gt_ref[...]
    xt = jnp.transpose(x)
    q = TR_NB // 4
    out_ref[...] = jnp.concatenate(
        [xt[0:q], xt[q:2 * q], xt[2 * q:3 * q], xt[3 * q:4 * q]], axis=1)


@functools.lru_cache(maxsize=1)
def _tc_relayout():
    return pl.pallas_call(
        _tr_body,
        grid=(TR_STEPS,),
        in_specs=[pl.BlockSpec((FEAT, TR_NB), lambda i: (0, i))],
        out_specs=pl.BlockSpec((TR_NB // 4, 128), lambda i: (i, 0)),
        out_shape=jax.ShapeDtypeStruct((TR_STEPS * TR_NB // 4, 128),
                                       jnp.float32),
        compiler_params=pltpu.CompilerParams(
            dimension_semantics=("arbitrary",)),
    )


def _tc_body(ro_ref, rd_ref, po_ref, pd_ref, io_ref, id_ref,
             wf_ref, bf_ref, wp_ref, bp_ref, oo_ref, od_ref):
    wf = wf_ref[...]
    bf = bf_ref[...]
    wp = wp_ref[...]
    bp = bp_ref[...]
    dn = (((0,), (1,)), ((), ()))

    for slot in range(SLOTS):
        for r_ref, p_ref, i_ref, out_ref in (
                (ro_ref, po_ref, io_ref, oo_ref),
                (rd_ref, pd_ref, id_ref, od_ref)):
            x = r_ref[:, 0, pl.ds(slot * BPW, BPW), :].reshape(B, FEAT)
            pe = p_ref[:, 0, pl.ds(slot * BPW, BPW), :].reshape(B, POS)
            f = lax.dot_general(wf, x, dn, preferred_element_type=jnp.float32)
            q = lax.dot_general(wp, pe, dn, preferred_element_type=jnp.float32)
            out_ref[slot] = (jnp.maximum(f + bf, 0.0)
                             + jnp.maximum(q + bp, 0.0) + i_ref[slot])


@functools.lru_cache(maxsize=1)
def _tc_mlp():
    rspec = lambda k: pl.BlockSpec((NW, 1, CHUNK, k), lambda j: (0, j, 0, 0))
    ispec = pl.BlockSpec((SLOTS, EMB, B), lambda j: (j, 0, 0))
    wspec = lambda a, b: pl.BlockSpec((a, b), lambda j: (0, 0))
    return pl.pallas_call(
        _tc_body,
        grid=(NCHUNK,),
        in_specs=[
            rspec(FEAT), rspec(FEAT), rspec(POS), rspec(POS),
            ispec, ispec,
            wspec(FEAT, EMB), wspec(EMB, 1), wspec(POS, EMB), wspec(EMB, 1),
        ],
        out_specs=[ispec, ispec],
        out_shape=[
            jax.ShapeDtypeStruct((N_OD, EMB, B), jnp.float32),
            jax.ShapeDtypeStruct((N_OD, EMB, B), jnp.float32),
        ],
        compiler_params=pltpu.CompilerParams(
            dimension_semantics=("parallel",)),
    )


def kernel(union_indices, o_positions, d_positions, position_encoding,
           iso_o, iso_d, grid_features, W_f, b_f, W_p, b_p):
    union_t = jnp.transpose(union_indices)
    opos_t = jnp.transpose(o_positions)
    dpos_t = jnp.transpose(d_positions)
    peflat = position_encoding.reshape(B * N_UNION, POS)

    grid_lin = _tc_relayout()(jnp.transpose(grid_features))
    grid_rm = grid_lin.reshape(TR_STEPS * TR_NB, FEAT)

    rows_o, rows_d, pe_o, pe_d = _sc_gather()(
        union_t, opos_t, dpos_t, peflat, grid_rm)

    iso_ot = jnp.transpose(iso_o, (1, 2, 0))
    iso_dt = jnp.transpose(iso_d, (1, 2, 0))
    oo, od = _tc_mlp()(
        rows_o, rows_d, pe_o, pe_d, iso_ot, iso_dt,
        W_f, b_f.reshape(EMB, 1), W_p, b_p.reshape(EMB, 1))
    return (jnp.transpose(oo, (2, 0, 1)), jnp.transpose(od, (2, 0, 1)))


# speedup vs baseline: 12.7763x; 1.3583x over previous
import functools

import jax
import jax.numpy as jnp
from jax import lax
from jax.experimental import pallas as pl
from jax.experimental.pallas import tpu as pltpu
from jax.experimental.pallas import tpu_sc as plsc

B = 1024
N_UNION = 200
N_OD = 100
FEAT = 32
POS = 8
EMB = 64

NC = 2
NS = 16
NW = NC * NS
BPW = B // NW
LANES = 16
CHUNK = 128
SLOTS = 4
NCHUNK = N_OD // SLOTS


def _sc_gather_build():
    mesh = plsc.VectorSubcoreMesh(core_axis_name="c", subcore_axis_name="s")

    @functools.partial(
        pl.kernel,
        out_type=jax.ShapeDtypeStruct((NW, NCHUNK, CHUNK, 128), jnp.float32),
        mesh=mesh,
        compiler_params=pltpu.CompilerParams(
            needs_layout_passes=False, use_tc_tiling_on_sc=False),
        scratch_types=[
            pltpu.VMEM((N_UNION, BPW), jnp.int32),
            pltpu.VMEM((2, N_OD, BPW), jnp.int32),
            pltpu.VMEM((2, N_OD * BPW), jnp.int32),
            pltpu.VMEM((2, N_OD * BPW), jnp.int32),
            pltpu.VMEM((2, CHUNK, FEAT), jnp.float32),
            pltpu.VMEM((2, CHUNK, POS), jnp.float32),
            pltpu.SemaphoreType.DMA,
        ],
    )
    def sc_gather(union_hbm, opos_hbm, dpos_hbm, peflat_hbm, grid_hbm,
                  stage_hbm,
                  union_v, pos_v, gidx_v, peidx_v, rows_v, pev, sem):
        wid = lax.axis_index("s") * NC + lax.axis_index("c")
        b0 = wid * BPW

        pltpu.sync_copy(union_hbm.at[:, pl.ds(b0, BPW)], union_v)
        pltpu.sync_copy(opos_hbm.at[:, pl.ds(b0, BPW)], pos_v.at[0])
        pltpu.sync_copy(dpos_hbm.at[:, pl.ds(b0, BPW)], pos_v.at[1])

        def compose(i, _):
            for e in range(2):
                for h in range(BPW // LANES):
                    db = h * LANES + lax.broadcasted_iota(jnp.int32, (LANES,), 0)
                    pv = pos_v[e, i, pl.ds(h * LANES, LANES)]
                    u = plsc.load_gather(union_v, [pv, db])
                    u = ((u & ~(TR_NB - 1)) | ((u & (TR_NB // 4 - 1)) << 2)
                         | ((u >> 11) & 3))
                    fl = i * BPW + h * LANES
                    gidx_v[e, pl.ds(fl, LANES)] = u
                    peidx_v[e, pl.ds(fl, LANES)] = (b0 + db) * N_UNION + pv
            return 0

        lax.fori_loop(0, N_OD, compose, 0)

        def gather_one(c, _):
            cps = [
                pltpu.async_copy(
                    grid_hbm.at[gidx_v.at[0, pl.ds(c * CHUNK, CHUNK)]],
                    rows_v.at[0], sem),
                pltpu.async_copy(
                    grid_hbm.at[gidx_v.at[1, pl.ds(c * CHUNK, CHUNK)]],
                    rows_v.at[1], sem),
                pltpu.async_copy(
                    peflat_hbm.at[peidx_v.at[0, pl.ds(c * CHUNK, CHUNK)]],
                    pev.at[0], sem),
                pltpu.async_copy(
                    peflat_hbm.at[peidx_v.at[1, pl.ds(c * CHUNK, CHUNK)]],
                    pev.at[1], sem),
            ]
            for cp in cps:
                cp.wait()
            pltpu.sync_copy(rows_v.at[0], stage_hbm.at[wid, c, :, pl.ds(0, FEAT)])
            pltpu.sync_copy(rows_v.at[1],
                            stage_hbm.at[wid, c, :, pl.ds(FEAT, FEAT)])
            pltpu.sync_copy(pev.at[0],
                            stage_hbm.at[wid, c, :, pl.ds(2 * FEAT, POS)])
            pltpu.sync_copy(pev.at[1],
                            stage_hbm.at[wid, c, :, pl.ds(2 * FEAT + POS, POS)])
            return 0

        lax.fori_loop(0, NCHUNK, gather_one, 0)

    return sc_gather


@functools.lru_cache(maxsize=1)
def _sc_gather():
    return _sc_gather_build()


GRID_N = 1000000
TR_NB = 8192
TR_STEPS = -(-GRID_N // TR_NB)


def _tr_body(gt_ref, out_ref):
    x = gt_ref[...]
    xt = jnp.transpose(x)
    q = TR_NB // 4
    out_ref[...] = jnp.concatenate(
        [xt[0:q], xt[q:2 * q], xt[2 * q:3 * q], xt[3 * q:4 * q]], axis=1)


@functools.lru_cache(maxsize=1)
def _tc_relayout():
    return pl.pallas_call(
        _tr_body,
        grid=(TR_STEPS,),
        in_specs=[pl.BlockSpec((FEAT, TR_NB), lambda i: (0, i))],
        out_specs=pl.BlockSpec((TR_NB // 4, 128), lambda i: (i, 0)),
        out_shape=jax.ShapeDtypeStruct((TR_STEPS * TR_NB // 4, 128),
                                       jnp.float32),
        compiler_params=pltpu.CompilerParams(
            dimension_semantics=("arbitrary",)),
    )


def _tc_body(st_ref, io_ref, id_ref,
             wf_ref, bf_ref, wp_ref, bp_ref, oo_ref, od_ref):
    wf = wf_ref[...]
    bf = bf_ref[...]
    wp = wp_ref[...]
    bp = bp_ref[...]
    dn = (((0,), (1,)), ((), ()))

    for slot in range(SLOTS):
        for e, (i_ref, out_ref) in enumerate(((io_ref, oo_ref),
                                              (id_ref, od_ref))):
            x = st_ref[:, 0, pl.ds(slot * BPW, BPW),
                       pl.ds(e * FEAT, FEAT)].reshape(B, FEAT)
            pe = st_ref[:, 0, pl.ds(slot * BPW, BPW),
                        pl.ds(2 * FEAT + e * POS, POS)].reshape(B, POS)
            f = lax.dot_general(wf, x, dn, preferred_element_type=jnp.float32)
            q = lax.dot_general(wp, pe, dn, preferred_element_type=jnp.float32)
            out_ref[slot] = (jnp.maximum(f + bf, 0.0)
                             + jnp.maximum(q + bp, 0.0) + i_ref[slot])


@functools.lru_cache(maxsize=1)
def _tc_mlp():
    ispec = pl.BlockSpec((SLOTS, EMB, B), lambda j: (j, 0, 0))
    wspec = lambda a, b: pl.BlockSpec((a, b), lambda j: (0, 0))
    return pl.pallas_call(
        _tc_body,
        grid=(NCHUNK,),
        in_specs=[
            pl.BlockSpec((NW, 1, CHUNK, 128), lambda j: (0, j, 0, 0)),
            ispec, ispec,
            wspec(FEAT, EMB), wspec(EMB, 1), wspec(POS, EMB), wspec(EMB, 1),
        ],
        out_specs=[ispec, ispec],
        out_shape=[
            jax.ShapeDtypeStruct((N_OD, EMB, B), jnp.float32),
            jax.ShapeDtypeStruct((N_OD, EMB, B), jnp.float32),
        ],
        compiler_params=pltpu.CompilerParams(
            dimension_semantics=("parallel",)),
    )


def kernel(union_indices, o_positions, d_positions, position_encoding,
           iso_o, iso_d, grid_features, W_f, b_f, W_p, b_p):
    union_t = jnp.transpose(union_indices)
    opos_t = jnp.transpose(o_positions)
    dpos_t = jnp.transpose(d_positions)
    peflat = position_encoding.reshape(B * N_UNION, POS)

    grid_lin = _tc_relayout()(jnp.transpose(grid_features))
    grid_rm = grid_lin.reshape(TR_STEPS * TR_NB, FEAT)

    stage = _sc_gather()(union_t, opos_t, dpos_t, peflat, grid_rm)

    iso_ot = jnp.transpose(iso_o, (1, 2, 0))
    iso_dt = jnp.transpose(iso_d, (1, 2, 0))
    oo, od = _tc_mlp()(
        stage, iso_ot, iso_dt,
        W_f, b_f.reshape(EMB, 1), W_p, b_p.reshape(EMB, 1))
    return (jnp.transpose(oo, (2, 0, 1)), jnp.transpose(od, (2, 0, 1)))


# speedup vs baseline: 12.9396x; 1.0128x over previous
import functools

import jax
import jax.numpy as jnp
from jax import lax
from jax.experimental import pallas as pl
from jax.experimental.pallas import tpu as pltpu
from jax.experimental.pallas import tpu_sc as plsc

B = 1024
N_UNION = 200
N_OD = 100
FEAT = 32
POS = 8
EMB = 64

NC = 2
NS = 16
NW = NC * NS
BPW = B // NW
LANES = 16
CHUNK = 128
SLOTS = 4
NCHUNK = N_OD // SLOTS


def _sc_gather_build():
    mesh = plsc.VectorSubcoreMesh(core_axis_name="c", subcore_axis_name="s")

    @functools.partial(
        pl.kernel,
        out_type=jax.ShapeDtypeStruct((NW, NCHUNK, CHUNK, 128), jnp.float32),
        mesh=mesh,
        compiler_params=pltpu.CompilerParams(
            needs_layout_passes=False, use_tc_tiling_on_sc=False),
        scratch_types=[
            pltpu.VMEM((N_UNION, BPW), jnp.int32),
            pltpu.VMEM((2, N_OD, BPW), jnp.int32),
            pltpu.VMEM((2, N_OD * BPW), jnp.int32),
            pltpu.VMEM((2, N_OD * BPW), jnp.int32),
            pltpu.VMEM((2, CHUNK, FEAT), jnp.float32),
            pltpu.VMEM((2, CHUNK, POS), jnp.float32),
            pltpu.SemaphoreType.DMA,
        ],
    )
    def sc_gather(union_hbm, opos_hbm, dpos_hbm, peflat_hbm, grid_hbm,
                  stage_hbm,
                  union_v, pos_v, gidx_v, peidx_v, rows_v, pev, sem):
        wid = lax.axis_index("s") * NC + lax.axis_index("c")
        b0 = wid * BPW

        pltpu.sync_copy(union_hbm.at[:, pl.ds(b0, BPW)], union_v)
        pltpu.sync_copy(opos_hbm.at[:, pl.ds(b0, BPW)], pos_v.at[0])
        pltpu.sync_copy(dpos_hbm.at[:, pl.ds(b0, BPW)], pos_v.at[1])

        def compose(i, _):
            for e in range(2):
                for h in range(BPW // LANES):
                    db = h * LANES + lax.broadcasted_iota(jnp.int32, (LANES,), 0)
                    pv = pos_v[e, i, pl.ds(h * LANES, LANES)]
                    u = plsc.load_gather(union_v, [pv, db])
                    u = ((u & ~(TR_NB - 1)) | ((u & (TR_NB // 4 - 1)) << 2)
                         | ((u >> TR_KSH) & 3))
                    fl = i * BPW + h * LANES
                    gidx_v[e, pl.ds(fl, LANES)] = u
                    peidx_v[e, pl.ds(fl, LANES)] = (b0 + db) * N_UNION + pv
            return 0

        lax.fori_loop(0, N_OD, compose, 0)

        def gather_one(c, _):
            cps = [
                pltpu.async_copy(
                    grid_hbm.at[gidx_v.at[0, pl.ds(c * CHUNK, CHUNK)]],
                    rows_v.at[0], sem),
                pltpu.async_copy(
                    grid_hbm.at[gidx_v.at[1, pl.ds(c * CHUNK, CHUNK)]],
                    rows_v.at[1], sem),
                pltpu.async_copy(
                    peflat_hbm.at[peidx_v.at[0, pl.ds(c * CHUNK, CHUNK)]],
                    pev.at[0], sem),
                pltpu.async_copy(
                    peflat_hbm.at[peidx_v.at[1, pl.ds(c * CHUNK, CHUNK)]],
                    pev.at[1], sem),
            ]
            for cp in cps:
                cp.wait()
            pltpu.sync_copy(rows_v.at[0], stage_hbm.at[wid, c, :, pl.ds(0, FEAT)])
            pltpu.sync_copy(rows_v.at[1],
                            stage_hbm.at[wid, c, :, pl.ds(FEAT, FEAT)])
            pltpu.sync_copy(pev.at[0],
                            stage_hbm.at[wid, c, :, pl.ds(2 * FEAT, POS)])
            pltpu.sync_copy(pev.at[1],
                            stage_hbm.at[wid, c, :, pl.ds(2 * FEAT + POS, POS)])
            return 0

        lax.fori_loop(0, NCHUNK, gather_one, 0)

    return sc_gather


@functools.lru_cache(maxsize=1)
def _sc_gather():
    return _sc_gather_build()


GRID_N = 1000000
TR_NB = 32768
TR_STEPS = -(-GRID_N // TR_NB)
TR_KSH = (TR_NB // 4).bit_length() - 1


def _tr_body(gt_ref, out_ref):
    x = gt_ref[...]
    xt = jnp.transpose(x)
    q = TR_NB // 4
    out_ref[...] = jnp.concatenate(
        [xt[0:q], xt[q:2 * q], xt[2 * q:3 * q], xt[3 * q:4 * q]], axis=1)


@functools.lru_cache(maxsize=1)
def _tc_relayout():
    return pl.pallas_call(
        _tr_body,
        grid=(TR_STEPS,),
        in_specs=[pl.BlockSpec((FEAT, TR_NB), lambda i: (0, i))],
        out_specs=pl.BlockSpec((TR_NB // 4, 128), lambda i: (i, 0)),
        out_shape=jax.ShapeDtypeStruct((TR_STEPS * TR_NB // 4, 128),
                                       jnp.float32),
        compiler_params=pltpu.CompilerParams(
            dimension_semantics=("arbitrary",)),
    )


def _tc_body(st_ref, io_ref, id_ref,
             wf_ref, bf_ref, wp_ref, bp_ref, oo_ref, od_ref):
    wf = wf_ref[...]
    bf = bf_ref[...]
    wp = wp_ref[...]
    bp = bp_ref[...]
    dn = (((0,), (1,)), ((), ()))

    for slot in range(SLOTS):
        for e, (i_ref, out_ref) in enumerate(((io_ref, oo_ref),
                                              (id_ref, od_ref))):
            x = st_ref[:, 0, pl.ds(slot * BPW, BPW),
                       pl.ds(e * FEAT, FEAT)].reshape(B, FEAT)
            pe = st_ref[:, 0, pl.ds(slot * BPW, BPW),
                        pl.ds(2 * FEAT + e * POS, POS)].reshape(B, POS)
            f = lax.dot_general(wf, x, dn, preferred_element_type=jnp.float32)
            q = lax.dot_general(wp, pe, dn, preferred_element_type=jnp.float32)
            out_ref[slot] = (jnp.maximum(f + bf, 0.0)
                             + jnp.maximum(q + bp, 0.0) + i_ref[slot])


@functools.lru_cache(maxsize=1)
def _tc_mlp():
    ispec = pl.BlockSpec((SLOTS, EMB, B), lambda j: (j, 0, 0))
    wspec = lambda a, b: pl.BlockSpec((a, b), lambda j: (0, 0))
    return pl.pallas_call(
        _tc_body,
        grid=(NCHUNK,),
        in_specs=[
            pl.BlockSpec((NW, 1, CHUNK, 128), lambda j: (0, j, 0, 0)),
            ispec, ispec,
            wspec(FEAT, EMB), wspec(EMB, 1), wspec(POS, EMB), wspec(EMB, 1),
        ],
        out_specs=[ispec, ispec],
        out_shape=[
            jax.ShapeDtypeStruct((N_OD, EMB, B), jnp.float32),
            jax.ShapeDtypeStruct((N_OD, EMB, B), jnp.float32),
        ],
        compiler_params=pltpu.CompilerParams(
            dimension_semantics=("parallel",)),
    )


def kernel(union_indices, o_positions, d_positions, position_encoding,
           iso_o, iso_d, grid_features, W_f, b_f, W_p, b_p):
    union_t = jnp.transpose(union_indices)
    opos_t = jnp.transpose(o_positions)
    dpos_t = jnp.transpose(d_positions)
    peflat = position_encoding.reshape(B * N_UNION, POS)

    grid_lin = _tc_relayout()(jnp.transpose(grid_features))
    grid_rm = grid_lin.reshape(TR_STEPS * TR_NB, FEAT)

    stage = _sc_gather()(union_t, opos_t, dpos_t, peflat, grid_rm)

    iso_ot = jnp.transpose(iso_o, (1, 2, 0))
    iso_dt = jnp.transpose(iso_d, (1, 2, 0))
    oo, od = _tc_mlp()(
        stage, iso_ot, iso_dt,
        W_f, b_f.reshape(EMB, 1), W_p, b_p.reshape(EMB, 1))
    return (jnp.transpose(oo, (2, 0, 1)), jnp.transpose(od, (2, 0, 1)))


# speedup vs baseline: 13.4680x; 1.0408x over previous
import functools

import jax
import jax.numpy as jnp
from jax import lax
from jax.experimental import pallas as pl
from jax.experimental.pallas import tpu as pltpu
from jax.experimental.pallas import tpu_sc as plsc

B = 1024
N_UNION = 200
N_OD = 100
FEAT = 32
POS = 8
EMB = 64

NC = 2
NS = 16
NW = NC * NS
BPW = B // NW
LANES = 16
CHUNK = 128
SLOTS = 4
NCHUNK = N_OD // SLOTS


def _sc_gather_build():
    mesh = plsc.VectorSubcoreMesh(core_axis_name="c", subcore_axis_name="s")

    @functools.partial(
        pl.kernel,
        out_type=jax.ShapeDtypeStruct((NW, NCHUNK, CHUNK, 128), jnp.float32),
        mesh=mesh,
        compiler_params=pltpu.CompilerParams(
            needs_layout_passes=False, use_tc_tiling_on_sc=False),
        scratch_types=[
            pltpu.VMEM((N_UNION, BPW), jnp.int32),
            pltpu.VMEM((2, N_OD, BPW), jnp.int32),
            pltpu.VMEM((2, N_OD * BPW), jnp.int32),
            pltpu.VMEM((2, N_OD * BPW), jnp.int32),
            pltpu.VMEM((2, CHUNK, FEAT), jnp.float32),
            pltpu.VMEM((2, CHUNK, POS), jnp.float32),
            pltpu.SemaphoreType.DMA,
        ],
    )
    def sc_gather(union_hbm, opos_hbm, dpos_hbm, peflat_hbm, grid_hbm,
                  stage_hbm,
                  union_v, pos_v, gidx_v, peidx_v, rows_v, pev, sem):
        wid = lax.axis_index("s") * NC + lax.axis_index("c")
        b0 = wid * BPW

        pltpu.sync_copy(union_hbm.at[:, pl.ds(b0, BPW)], union_v)
        pltpu.sync_copy(opos_hbm.at[:, pl.ds(b0, BPW)], pos_v.at[0])
        pltpu.sync_copy(dpos_hbm.at[:, pl.ds(b0, BPW)], pos_v.at[1])

        def compose(i, _):
            for e in range(2):
                for h in range(BPW // LANES):
                    db = h * LANES + lax.broadcasted_iota(jnp.int32, (LANES,), 0)
                    pv = pos_v[e, i, pl.ds(h * LANES, LANES)]
                    u = plsc.load_gather(union_v, [pv, db])
                    u = (u & ~127) | ((u & 31) << 2) | ((u >> 5) & 3)
                    fl = i * BPW + h * LANES
                    gidx_v[e, pl.ds(fl, LANES)] = u
                    bb = b0 + db
                    peidx_v[e, pl.ds(fl, LANES)] = (
                        pv * 1024 + ((bb & 63) << 4) + (bb >> 6))
            return 0

        lax.fori_loop(0, N_OD, compose, 0)

        def gather_one(c, _):
            cps = [
                pltpu.async_copy(
                    grid_hbm.at[gidx_v.at[0, pl.ds(c * CHUNK, CHUNK)]],
                    rows_v.at[0], sem),
                pltpu.async_copy(
                    grid_hbm.at[gidx_v.at[1, pl.ds(c * CHUNK, CHUNK)]],
                    rows_v.at[1], sem),
                pltpu.async_copy(
                    peflat_hbm.at[peidx_v.at[0, pl.ds(c * CHUNK, CHUNK)]],
                    pev.at[0], sem),
                pltpu.async_copy(
                    peflat_hbm.at[peidx_v.at[1, pl.ds(c * CHUNK, CHUNK)]],
                    pev.at[1], sem),
            ]
            for cp in cps:
                cp.wait()
            pltpu.sync_copy(rows_v.at[0], stage_hbm.at[wid, c, :, pl.ds(0, FEAT)])
            pltpu.sync_copy(rows_v.at[1],
                            stage_hbm.at[wid, c, :, pl.ds(FEAT, FEAT)])
            pltpu.sync_copy(pev.at[0],
                            stage_hbm.at[wid, c, :, pl.ds(2 * FEAT, POS)])
            pltpu.sync_copy(pev.at[1],
                            stage_hbm.at[wid, c, :, pl.ds(2 * FEAT + POS, POS)])
            return 0

        lax.fori_loop(0, NCHUNK, gather_one, 0)

    return sc_gather


@functools.lru_cache(maxsize=1)
def _sc_gather():
    return _sc_gather_build()


GRID_N = 1000000
TR_NB = 32768
TR_STEPS = -(-GRID_N // TR_NB)
TR_KSH = (TR_NB // 4).bit_length() - 1


def _tr_body(gt_ref, out_ref):
    for j in range(TR_NB // 128):
        xt = jnp.transpose(gt_ref[:, pl.ds(j * 128, 128)])
        out_ref[pl.ds(j * 32, 32), :] = jnp.concatenate(
            [xt[0:32], xt[32:64], xt[64:96], xt[96:128]], axis=1)


@functools.lru_cache(maxsize=1)
def _tc_relayout():
    return pl.pallas_call(
        _tr_body,
        grid=(TR_STEPS,),
        in_specs=[pl.BlockSpec((FEAT, TR_NB), lambda i: (0, i))],
        out_specs=pl.BlockSpec((TR_NB // 4, 128), lambda i: (i, 0)),
        out_shape=jax.ShapeDtypeStruct((TR_STEPS * TR_NB // 4, 128),
                                       jnp.float32),
        compiler_params=pltpu.CompilerParams(
            dimension_semantics=("arbitrary",)),
    )


PE_PB = 25


def _pe_body(pc_ref, out_ref):
    for p in range(PE_PB):
        xt = jnp.transpose(pc_ref[pl.ds(p * POS, POS), :])
        out_ref[pl.ds(p * 64, 64), :] = jnp.concatenate(
            [xt[k * 64:(k + 1) * 64] for k in range(16)], axis=1)


@functools.lru_cache(maxsize=1)
def _pe_relayout():
    return pl.pallas_call(
        _pe_body,
        grid=(N_UNION // PE_PB,),
        in_specs=[pl.BlockSpec((PE_PB * POS, B), lambda i: (i, 0))],
        out_specs=pl.BlockSpec((PE_PB * 64, 128), lambda i: (i, 0)),
        out_shape=jax.ShapeDtypeStruct((N_UNION * 64, 128), jnp.float32),
        compiler_params=pltpu.CompilerParams(
            dimension_semantics=("arbitrary",)),
    )


def _tc_body(st_ref, io_ref, id_ref,
             wf_ref, bf_ref, wp_ref, bp_ref, oo_ref, od_ref):
    wf = wf_ref[...]
    bf = bf_ref[...]
    wp = wp_ref[...]
    bp = bp_ref[...]
    dn = (((0,), (1,)), ((), ()))

    for slot in range(SLOTS):
        for e, (i_ref, out_ref) in enumerate(((io_ref, oo_ref),
                                              (id_ref, od_ref))):
            x = st_ref[:, 0, pl.ds(slot * BPW, BPW),
                       pl.ds(e * FEAT, FEAT)].reshape(B, FEAT)
            pe = st_ref[:, 0, pl.ds(slot * BPW, BPW),
                        pl.ds(2 * FEAT + e * POS, POS)].reshape(B, POS)
            f = lax.dot_general(wf, x, dn, preferred_element_type=jnp.float32)
            q = lax.dot_general(wp, pe, dn, preferred_element_type=jnp.float32)
            out_ref[slot] = (jnp.maximum(f + bf, 0.0)
                             + jnp.maximum(q + bp, 0.0) + i_ref[slot])


@functools.lru_cache(maxsize=1)
def _tc_mlp():
    ispec = pl.BlockSpec((SLOTS, EMB, B), lambda j: (j, 0, 0))
    wspec = lambda a, b: pl.BlockSpec((a, b), lambda j: (0, 0))
    return pl.pallas_call(
        _tc_body,
        grid=(NCHUNK,),
        in_specs=[
            pl.BlockSpec((NW, 1, CHUNK, 128), lambda j: (0, j, 0, 0)),
            ispec, ispec,
            wspec(FEAT, EMB), wspec(EMB, 1), wspec(POS, EMB), wspec(EMB, 1),
        ],
        out_specs=[ispec, ispec],
        out_shape=[
            jax.ShapeDtypeStruct((N_OD, EMB, B), jnp.float32),
            jax.ShapeDtypeStruct((N_OD, EMB, B), jnp.float32),
        ],
        compiler_params=pltpu.CompilerParams(
            dimension_semantics=("parallel",)),
    )


def kernel(union_indices, o_positions, d_positions, position_encoding,
           iso_o, iso_d, grid_features, W_f, b_f, W_p, b_p):
    union_t = jnp.transpose(union_indices)
    opos_t = jnp.transpose(o_positions)
    dpos_t = jnp.transpose(d_positions)
    pe_packed = _pe_relayout()(
        jnp.transpose(position_encoding, (1, 2, 0)).reshape(
            N_UNION * POS, B))
    peflat = pe_packed.reshape(N_UNION * B, POS)

    grid_lin = _tc_relayout()(jnp.transpose(grid_features))
    grid_rm = grid_lin.reshape(TR_STEPS * TR_NB, FEAT)

    stage = _sc_gather()(union_t, opos_t, dpos_t, peflat, grid_rm)

    iso_ot = jnp.transpose(iso_o, (1, 2, 0))
    iso_dt = jnp.transpose(iso_d, (1, 2, 0))
    oo, od = _tc_mlp()(
        stage, iso_ot, iso_dt,
        W_f, b_f.reshape(EMB, 1), W_p, b_p.reshape(EMB, 1))
    return (jnp.transpose(oo, (2, 0, 1)), jnp.transpose(od, (2, 0, 1)))


# speedup vs baseline: 13.8207x; 1.0262x over previous
import functools

import jax
import jax.numpy as jnp
from jax import lax
from jax.experimental import pallas as pl
from jax.experimental.pallas import tpu as pltpu
from jax.experimental.pallas import tpu_sc as plsc

B = 1024
N_UNION = 200
N_OD = 100
FEAT = 32
POS = 8
EMB = 64

NC = 2
NS = 16
NW = NC * NS
BPW = B // NW
LANES = 16
CHUNK = 128
SLOTS = 4
NCHUNK = N_OD // SLOTS


def _sc_compose_build():
    mesh = plsc.VectorSubcoreMesh(core_axis_name="c", subcore_axis_name="s")

    @functools.partial(
        pl.kernel,
        out_type=(
            jax.ShapeDtypeStruct((NW, 2, N_OD * BPW), jnp.int32),
            jax.ShapeDtypeStruct((NW, NCHUNK, CHUNK, 2 * POS), jnp.float32),
        ),
        mesh=mesh,
        compiler_params=pltpu.CompilerParams(
            needs_layout_passes=False, use_tc_tiling_on_sc=False),
        scratch_types=[
            pltpu.VMEM((N_UNION, BPW), jnp.int32),
            pltpu.VMEM((2, N_OD, BPW), jnp.int32),
            pltpu.VMEM((2, N_OD * BPW), jnp.int32),
            pltpu.VMEM((2, N_OD * BPW), jnp.int32),
            pltpu.VMEM((2, CHUNK, POS), jnp.float32),
            pltpu.SemaphoreType.DMA,
        ],
    )
    def sc_compose(union_hbm, opos_hbm, dpos_hbm, peflat_hbm,
                   gidx_hbm, pstage_hbm,
                   union_v, pos_v, gidx_v, peidx_v, pev, sem):
        wid = lax.axis_index("s") * NC + lax.axis_index("c")
        b0 = wid * BPW

        pltpu.sync_copy(union_hbm.at[:, pl.ds(b0, BPW)], union_v)
        pltpu.sync_copy(opos_hbm.at[:, pl.ds(b0, BPW)], pos_v.at[0])
        pltpu.sync_copy(dpos_hbm.at[:, pl.ds(b0, BPW)], pos_v.at[1])

        def compose(i, _):
            for e in range(2):
                for h in range(BPW // LANES):
                    db = h * LANES + lax.broadcasted_iota(jnp.int32, (LANES,), 0)
                    pv = pos_v[e, i, pl.ds(h * LANES, LANES)]
                    u = plsc.load_gather(union_v, [pv, db])
                    u = (u & ~127) | ((u & 31) << 2) | ((u >> 5) & 3)
                    fl = i * BPW + h * LANES
                    gidx_v[e, pl.ds(fl, LANES)] = u
                    bb = b0 + db
                    peidx_v[e, pl.ds(fl, LANES)] = (
                        pv * 1024 + ((bb & 63) << 4) + (bb >> 6))
            return 0

        lax.fori_loop(0, N_OD, compose, 0)
        pltpu.sync_copy(gidx_v, gidx_hbm.at[wid])

        def pe_gather(c, _):
            cps = [
                pltpu.async_copy(
                    peflat_hbm.at[peidx_v.at[0, pl.ds(c * CHUNK, CHUNK)]],
                    pev.at[0], sem),
                pltpu.async_copy(
                    peflat_hbm.at[peidx_v.at[1, pl.ds(c * CHUNK, CHUNK)]],
                    pev.at[1], sem),
            ]
            for cp in cps:
                cp.wait()
            pltpu.sync_copy(pev.at[0], pstage_hbm.at[wid, c, :, pl.ds(0, POS)])
            pltpu.sync_copy(pev.at[1],
                            pstage_hbm.at[wid, c, :, pl.ds(POS, POS)])
            return 0

        lax.fori_loop(0, NCHUNK, pe_gather, 0)

    return sc_compose


def _sc_gather_build():
    mesh = plsc.VectorSubcoreMesh(core_axis_name="c", subcore_axis_name="s")

    @functools.partial(
        pl.kernel,
        out_type=jax.ShapeDtypeStruct((NW, NCHUNK, CHUNK, 128), jnp.float32),
        mesh=mesh,
        compiler_params=pltpu.CompilerParams(
            needs_layout_passes=False, use_tc_tiling_on_sc=False),
        scratch_types=[
            pltpu.VMEM((2, N_OD * BPW), jnp.int32),
            pltpu.VMEM((2, CHUNK, FEAT), jnp.float32),
            pltpu.VMEM((CHUNK, 2 * POS), jnp.float32),
            pltpu.SemaphoreType.DMA,
        ],
    )
    def sc_gather(gidx_hbm, pstage_hbm, grid_hbm, stage_hbm,
                  gidx_v, rows_v, pev, sem):
        wid = lax.axis_index("s") * NC + lax.axis_index("c")
        pltpu.sync_copy(gidx_hbm.at[wid], gidx_v)

        def gather_one(c, _):
            cps = [
                pltpu.async_copy(
                    grid_hbm.at[gidx_v.at[0, pl.ds(c * CHUNK, CHUNK)]],
                    rows_v.at[0], sem),
                pltpu.async_copy(
                    grid_hbm.at[gidx_v.at[1, pl.ds(c * CHUNK, CHUNK)]],
                    rows_v.at[1], sem),
                pltpu.async_copy(pstage_hbm.at[wid, c], pev, sem),
            ]
            for cp in cps:
                cp.wait()
            pltpu.sync_copy(rows_v.at[0], stage_hbm.at[wid, c, :, pl.ds(0, FEAT)])
            pltpu.sync_copy(rows_v.at[1],
                            stage_hbm.at[wid, c, :, pl.ds(FEAT, FEAT)])
            pltpu.sync_copy(pev, stage_hbm.at[wid, c, :, pl.ds(2 * FEAT, 2 * POS)])
            return 0

        lax.fori_loop(0, NCHUNK, gather_one, 0)

    return sc_gather


@functools.lru_cache(maxsize=1)
def _sc_compose():
    return _sc_compose_build()


@functools.lru_cache(maxsize=1)
def _sc_gather():
    return _sc_gather_build()


GRID_N = 1000000
TR_NB = 32768
TR_STEPS = -(-GRID_N // TR_NB)
TR_KSH = (TR_NB // 4).bit_length() - 1


def _tr_body(gt_ref, out_ref):
    for j in range(TR_NB // 128):
        xt = jnp.transpose(gt_ref[:, pl.ds(j * 128, 128)])
        out_ref[pl.ds(j * 32, 32), :] = jnp.concatenate(
            [xt[0:32], xt[32:64], xt[64:96], xt[96:128]], axis=1)


@functools.lru_cache(maxsize=1)
def _tc_relayout():
    return pl.pallas_call(
        _tr_body,
        grid=(TR_STEPS,),
        in_specs=[pl.BlockSpec((FEAT, TR_NB), lambda i: (0, i))],
        out_specs=pl.BlockSpec((TR_NB // 4, 128), lambda i: (i, 0)),
        out_shape=jax.ShapeDtypeStruct((TR_STEPS * TR_NB // 4, 128),
                                       jnp.float32),
        compiler_params=pltpu.CompilerParams(
            dimension_semantics=("arbitrary",)),
    )


PE_PB = 25


def _pe_body(pc_ref, out_ref):
    for p in range(PE_PB):
        xt = jnp.transpose(pc_ref[pl.ds(p * POS, POS), :])
        out_ref[pl.ds(p * 64, 64), :] = jnp.concatenate(
            [xt[k * 64:(k + 1) * 64] for k in range(16)], axis=1)


@functools.lru_cache(maxsize=1)
def _pe_relayout():
    return pl.pallas_call(
        _pe_body,
        grid=(N_UNION // PE_PB,),
        in_specs=[pl.BlockSpec((PE_PB * POS, B), lambda i: (i, 0))],
        out_specs=pl.BlockSpec((PE_PB * 64, 128), lambda i: (i, 0)),
        out_shape=jax.ShapeDtypeStruct((N_UNION * 64, 128), jnp.float32),
        compiler_params=pltpu.CompilerParams(
            dimension_semantics=("arbitrary",)),
    )


def _tc_body(st_ref, io_ref, id_ref,
             wf_ref, bf_ref, wp_ref, bp_ref, oo_ref, od_ref):
    wf = wf_ref[...]
    bf = bf_ref[...]
    wp = wp_ref[...]
    bp = bp_ref[...]
    dn = (((0,), (1,)), ((), ()))

    for slot in range(SLOTS):
        for e, (i_ref, out_ref) in enumerate(((io_ref, oo_ref),
                                              (id_ref, od_ref))):
            x = st_ref[:, 0, pl.ds(slot * BPW, BPW),
                       pl.ds(e * FEAT, FEAT)].reshape(B, FEAT)
            pe = st_ref[:, 0, pl.ds(slot * BPW, BPW),
                        pl.ds(2 * FEAT + e * POS, POS)].reshape(B, POS)
            f = lax.dot_general(wf, x, dn, preferred_element_type=jnp.float32)
            q = lax.dot_general(wp, pe, dn, preferred_element_type=jnp.float32)
            out_ref[slot] = (jnp.maximum(f + bf, 0.0)
                             + jnp.maximum(q + bp, 0.0) + i_ref[slot])


@functools.lru_cache(maxsize=1)
def _tc_mlp():
    ispec = pl.BlockSpec((SLOTS, EMB, B), lambda j: (j, 0, 0))
    wspec = lambda a, b: pl.BlockSpec((a, b), lambda j: (0, 0))
    return pl.pallas_call(
        _tc_body,
        grid=(NCHUNK,),
        in_specs=[
            pl.BlockSpec((NW, 1, CHUNK, 128), lambda j: (0, j, 0, 0)),
            ispec, ispec,
            wspec(FEAT, EMB), wspec(EMB, 1), wspec(POS, EMB), wspec(EMB, 1),
        ],
        out_specs=[ispec, ispec],
        out_shape=[
            jax.ShapeDtypeStruct((N_OD, EMB, B), jnp.float32),
            jax.ShapeDtypeStruct((N_OD, EMB, B), jnp.float32),
        ],
        compiler_params=pltpu.CompilerParams(
            dimension_semantics=("parallel",)),
    )


def kernel(union_indices, o_positions, d_positions, position_encoding,
           iso_o, iso_d, grid_features, W_f, b_f, W_p, b_p):
    union_t = jnp.transpose(union_indices)
    opos_t = jnp.transpose(o_positions)
    dpos_t = jnp.transpose(d_positions)
    pe_packed = _pe_relayout()(
        jnp.transpose(position_encoding, (1, 2, 0)).reshape(
            N_UNION * POS, B))
    peflat = pe_packed.reshape(N_UNION * B, POS)

    grid_lin = _tc_relayout()(jnp.transpose(grid_features))
    grid_rm = grid_lin.reshape(TR_STEPS * TR_NB, FEAT)

    gidx, pstage = _sc_compose()(union_t, opos_t, dpos_t, peflat)
    stage = _sc_gather()(gidx, pstage, grid_rm)

    iso_ot = jnp.transpose(iso_o, (1, 2, 0))
    iso_dt = jnp.transpose(iso_d, (1, 2, 0))
    oo, od = _tc_mlp()(
        stage, iso_ot, iso_dt,
        W_f, b_f.reshape(EMB, 1), W_p, b_p.reshape(EMB, 1))
    return (jnp.transpose(oo, (2, 0, 1)), jnp.transpose(od, (2, 0, 1)))
